# baseline (speedup 1.0000x reference)
import functools

import jax
import jax.numpy as jnp
from jax import lax
from jax.experimental import pallas as pl
from jax.experimental.pallas import tpu as pltpu
from jax.experimental.pallas import tpu_sc as plsc

N_ATOMS = 4194304
TABLE_LEN = 119
L = 16
NC = 2
NS = 16
NW = NC * NS
PER_W = N_ATOMS // NW
CHUNK = 8192
NCHUNK = PER_W // CHUNK

_mesh = plsc.VectorSubcoreMesh(core_axis_name="c", subcore_axis_name="s")


@functools.partial(
    pl.kernel,
    mesh=_mesh,
    out_type=jax.ShapeDtypeStruct((N_ATOMS,), jnp.float32),
    scratch_types=[
        pltpu.VMEM((TABLE_LEN,), jnp.float32),
        pltpu.VMEM((CHUNK,), jnp.int32),
        pltpu.VMEM((CHUNK,), jnp.int32),
        pltpu.VMEM((CHUNK,), jnp.float32),
        pltpu.VMEM((CHUNK,), jnp.float32),
        pltpu.SemaphoreType.DMA,
        pltpu.SemaphoreType.DMA,
        pltpu.SemaphoreType.DMA,
        pltpu.SemaphoreType.DMA,
    ],
    compiler_params=pltpu.CompilerParams(needs_layout_passes=False),
)
def _lookup(species_hbm, const_hbm, out_hbm,
            table_v, idx0, idx1, val0, val1,
            sin0, sin1, sout0, sout1):
    wid = lax.axis_index("s") * NC + lax.axis_index("c")
    base = wid * PER_W

    pltpu.sync_copy(const_hbm, table_v)

    idx = (idx0, idx1)
    val = (val0, val1)
    sin = (sin0, sin1)
    sout = (sout0, sout1)

    def in_copy(c, b):
        return pltpu.make_async_copy(
            species_hbm.at[pl.ds(base + c * CHUNK, CHUNK)], idx[b], sin[b])

    def out_copy(c, b):
        return pltpu.make_async_copy(
            val[b], out_hbm.at[pl.ds(base + c * CHUNK, CHUNK)], sout[b])

    def compute(b):
        idx_ref = idx[b]
        val_ref = val[b]

        @plsc.parallel_loop(0, CHUNK, L, unroll=8)
        def _(i):
            sl = pl.ds(i, L)
            val_ref[sl] = plsc.load_gather(table_v, [idx_ref[sl]])

    in_copy(0, 0).start()
    in_copy(1, 1).start()

    @pl.loop(0, NCHUNK, step=2)
    def _(g):
        for b in range(2):
            c = g + b
            in_copy(c, b).wait()

            @pl.when(g >= 2)
            def _():
                out_copy(c - 2, b).wait()

            compute(b)
            out_copy(c, b).start()

            @pl.when(g < NCHUNK - 2)
            def _():
                in_copy(c + 2, b).start()

    out_copy(NCHUNK - 2, 0).wait()
    out_copy(NCHUNK - 1, 1).wait()


def kernel(species, constant):
    return _lookup(species, constant)


# speedup vs baseline: 1.0446x; 1.0446x over previous
import functools

import jax
import jax.numpy as jnp
from jax import lax
from jax.experimental import pallas as pl
from jax.experimental.pallas import tpu as pltpu
from jax.experimental.pallas import tpu_sc as plsc

N_ATOMS = 4194304
TABLE_LEN = 119
L = 16
NC = 2
NS = 16
NW = NC * NS
PER_W = N_ATOMS // NW
CHUNK = 16384
NCHUNK = PER_W // CHUNK

_mesh = plsc.VectorSubcoreMesh(core_axis_name="c", subcore_axis_name="s")


@functools.partial(
    pl.kernel,
    mesh=_mesh,
    out_type=jax.ShapeDtypeStruct((N_ATOMS,), jnp.float32),
    scratch_types=[
        pltpu.VMEM((TABLE_LEN,), jnp.float32),
        pltpu.VMEM((CHUNK,), jnp.int32),
        pltpu.VMEM((CHUNK,), jnp.int32),
        pltpu.VMEM((CHUNK,), jnp.float32),
        pltpu.VMEM((CHUNK,), jnp.float32),
        pltpu.SemaphoreType.DMA,
        pltpu.SemaphoreType.DMA,
        pltpu.SemaphoreType.DMA,
        pltpu.SemaphoreType.DMA,
    ],
    compiler_params=pltpu.CompilerParams(needs_layout_passes=False),
)
def _lookup(species_hbm, const_hbm, out_hbm,
            table_v, idx0, idx1, val0, val1,
            sin0, sin1, sout0, sout1):
    wid = lax.axis_index("s") * NC + lax.axis_index("c")
    base = wid * PER_W

    pltpu.sync_copy(const_hbm, table_v)

    idx = (idx0, idx1)
    val = (val0, val1)
    sin = (sin0, sin1)
    sout = (sout0, sout1)

    def in_copy(c, b):
        return pltpu.make_async_copy(
            species_hbm.at[pl.ds(base + c * CHUNK, CHUNK)], idx[b], sin[b])

    def out_copy(c, b):
        return pltpu.make_async_copy(
            val[b], out_hbm.at[pl.ds(base + c * CHUNK, CHUNK)], sout[b])

    def compute(b):
        idx_ref = idx[b]
        val_ref = val[b]

        @plsc.parallel_loop(0, CHUNK, L, unroll=8)
        def _(i):
            sl = pl.ds(i, L)
            val_ref[sl] = plsc.load_gather(table_v, [idx_ref[sl]])

    in_copy(0, 0).start()
    in_copy(1, 1).start()

    @pl.loop(0, NCHUNK, step=2)
    def _(g):
        for b in range(2):
            c = g + b
            in_copy(c, b).wait()

            @pl.when(g >= 2)
            def _():
                out_copy(c - 2, b).wait()

            compute(b)
            out_copy(c, b).start()

            @pl.when(g < NCHUNK - 2)
            def _():
                in_copy(c + 2, b).start()

    out_copy(NCHUNK - 2, 0).wait()
    out_copy(NCHUNK - 1, 1).wait()


def kernel(species, constant):
    return _lookup(species, constant)


# speedup vs baseline: 1.0801x; 1.0340x over previous
import functools

import jax
import jax.numpy as jnp
from jax import lax
from jax.experimental import pallas as pl
from jax.experimental.pallas import tpu as pltpu
from jax.experimental.pallas import tpu_sc as plsc

N_ATOMS = 4194304
TABLE_LEN = 119
L = 16
NC = 2
NS = 16
NW = NC * NS
PER_W = N_ATOMS // NW
CHUNK = 16384
NCHUNK = PER_W // CHUNK

_mesh = plsc.VectorSubcoreMesh(core_axis_name="c", subcore_axis_name="s")


@functools.partial(
    pl.kernel,
    mesh=_mesh,
    out_type=jax.ShapeDtypeStruct((N_ATOMS,), jnp.float32),
    scratch_types=[
        pltpu.VMEM((TABLE_LEN,), jnp.float32),
        pltpu.VMEM((CHUNK,), jnp.int32),
        pltpu.VMEM((CHUNK,), jnp.int32),
        pltpu.VMEM((CHUNK,), jnp.float32),
        pltpu.VMEM((CHUNK,), jnp.float32),
        pltpu.SemaphoreType.DMA,
        pltpu.SemaphoreType.DMA,
        pltpu.SemaphoreType.DMA,
        pltpu.SemaphoreType.DMA,
    ],
    compiler_params=pltpu.CompilerParams(needs_layout_passes=False),
)
def _lookup(species_hbm, const_hbm, out_hbm,
            table_v, idx0, idx1, val0, val1,
            sin0, sin1, sout0, sout1):
    wid = lax.axis_index("s") * NC + lax.axis_index("c")
    base = wid * PER_W

    idx = (idx0, idx1)
    val = (val0, val1)
    sin = (sin0, sin1)
    sout = (sout0, sout1)

    def in_copy(c, b):
        return pltpu.make_async_copy(
            species_hbm.at[pl.ds(base + c * CHUNK, CHUNK)], idx[b], sin[b])

    def out_copy(c, b):
        return pltpu.make_async_copy(
            val[b], out_hbm.at[pl.ds(base + c * CHUNK, CHUNK)], sout[b])

    def compute(b):
        idx_ref = idx[b]
        val_ref = val[b]

        @plsc.parallel_loop(0, CHUNK, L, unroll=8)
        def _(i):
            sl = pl.ds(i, L)
            val_ref[sl] = plsc.load_gather(table_v, [idx_ref[sl]])

    in_copy(0, 0).start()
    in_copy(1, 1).start()
    pltpu.sync_copy(const_hbm, table_v)

    @pl.loop(0, NCHUNK, step=2)
    def _(g):
        for b in range(2):
            c = g + b
            in_copy(c, b).wait()

            @pl.when(g >= 2)
            def _():
                out_copy(c - 2, b).wait()

            compute(b)
            out_copy(c, b).start()

            @pl.when(g < NCHUNK - 2)
            def _():
                in_copy(c + 2, b).start()

    out_copy(NCHUNK - 2, 0).wait()
    out_copy(NCHUNK - 1, 1).wait()


def kernel(species, constant):
    return _lookup(species, constant)
